# Initial kernel scaffold; baseline (speedup 1.0000x reference)
import jax
import jax.numpy as jnp
from jax.experimental import pallas as pl


def kernel(x, edge_index, edge_attr, batch, W1, b1, W2, b2, W_root, gamma, beta):
    raise NotImplementedError("write your pallas kernel here")


# speedup vs baseline: 5.9177x; 5.9177x over previous
import functools

import jax
import jax.numpy as jnp
from jax import lax
from jax.experimental import pallas as pl
from jax.experimental.pallas import tpu as pltpu
from jax.experimental.pallas import tpu_sc as plsc

N = 10000
E = 320000
C_IN = 128
C_OUT = 128

NC = 2
NS = 16
NW = NC * NS
EPW = E // NW
CH = 80
NCHUNK = EPW // CH
NPAD = 10240
RPT = NPAD // NS


def _precompute_body(x_ref, w1_ref, b1_ref, wr_ref, p_ref, q_ref, r_ref):
    xb = x_ref[...]
    p_ref[...] = (jnp.dot(xb, w1_ref[0:C_IN, :],
                          preferred_element_type=jnp.float32) + b1_ref[...])
    q_ref[...] = jnp.dot(xb, w1_ref[C_IN:2 * C_IN, :],
                         preferred_element_type=jnp.float32)
    r_ref[...] = jnp.dot(xb, wr_ref[...], preferred_element_type=jnp.float32)


def _precompute(x, W1, b1, W_root):
    blk = 2000
    grid = N // blk
    out = jax.ShapeDtypeStruct((N, C_OUT), jnp.float32)
    return pl.pallas_call(
        _precompute_body,
        grid=(grid,),
        in_specs=[
            pl.BlockSpec((blk, C_IN), lambda i: (i, 0)),
            pl.BlockSpec((2 * C_IN, C_OUT), lambda i: (0, 0)),
            pl.BlockSpec((1, C_OUT), lambda i: (0, 0)),
            pl.BlockSpec((C_IN, C_OUT), lambda i: (0, 0)),
        ],
        out_specs=[
            pl.BlockSpec((blk, C_OUT), lambda i: (i, 0)),
            pl.BlockSpec((blk, C_OUT), lambda i: (i, 0)),
            pl.BlockSpec((blk, C_OUT), lambda i: (i, 0)),
        ],
        out_shape=[out, out, out],
    )(x, W1, b1.reshape(1, C_OUT), W_root)


def _edge_body(p_hbm, q_hbm, src_hbm, dst_hbm, s_out, cnt_parts,
               didx, sidx, pbuf, qbuf, clocal, sem):
    c = lax.axis_index("c")
    s = lax.axis_index("s")
    w = c * NS + s

    zvec = jnp.zeros((16,), jnp.float32)

    def _zero_row(i, _):
        for j in range(C_OUT // 16):
            pbuf[i, pl.ds(j * 16, 16)] = zvec
        return 0

    lax.fori_loop(0, CH, _zero_row, 0)

    def _zfill(k, _):
        pltpu.sync_copy(pbuf, s_out.at[pl.ds(s * RPT + k * CH, CH)])
        return 0

    lax.fori_loop(0, RPT // CH, _zfill, 0)

    def _zcnt(i, _):
        clocal[pl.ds(i * 16, 16)] = zvec
        return 0

    lax.fori_loop(0, NPAD // 16, _zcnt, 0)
    plsc.subcore_barrier()

    ones16 = jnp.ones((16,), jnp.float32)

    def _chunk(j, _):
        base = w * EPW + j * CH
        pltpu.sync_copy(dst_hbm.at[pl.ds(base, CH)], didx)
        pltpu.sync_copy(src_hbm.at[pl.ds(base, CH)], sidx)
        cp_p = pltpu.async_copy(p_hbm.at[didx], pbuf, sem)
        cp_q = pltpu.async_copy(q_hbm.at[sidx], qbuf, sem)
        cp_p.wait()
        cp_q.wait()

        def _relu_row(r, _):
            for jj in range(C_OUT // 16):
                sl = pl.ds(jj * 16, 16)
                pbuf[r, sl] = jnp.maximum(pbuf[r, sl] + qbuf[r, sl], 0.0)
            return 0

        lax.fori_loop(0, CH, _relu_row, 0)
        pltpu.sync_copy(pbuf, s_out.at[didx], add=True)
        for k in range(CH // 16):
            idx16 = didx[pl.ds(k * 16, 16)]
            plsc.addupdate_scatter(clocal, [idx16], ones16)
        return 0

    lax.fori_loop(0, NCHUNK, _chunk, 0)
    pltpu.sync_copy(clocal, cnt_parts.at[s])
    plsc.subcore_barrier()


def _edge_kernel_body(p_hbm, q_hbm, src_hbm, dst_hbm, s_hbm, cnt_hbm,
                      s_sh, cnt_parts, didx, sidx, pbuf, qbuf, clocal,
                      cwork, cvec, sem):
    c = lax.axis_index("c")
    s = lax.axis_index("s")
    _edge_body(p_hbm, q_hbm, src_hbm, dst_hbm, s_sh, cnt_parts,
               didx, sidx, pbuf, qbuf, clocal, sem)

    def _publish(k, _):
        base = s * RPT + k * CH
        pltpu.sync_copy(s_sh.at[pl.ds(base, CH)], pbuf)
        pltpu.sync_copy(pbuf, s_hbm.at[c, pl.ds(base, CH)])
        return 0

    lax.fori_loop(0, RPT // CH, _publish, 0)

    def _czero(i, _):
        cvec[pl.ds(i * 16, 16)] = jnp.zeros((16,), jnp.float32)
        return 0

    lax.fori_loop(0, RPT // 16, _czero, 0)
    for t in range(NS):
        pltpu.sync_copy(cnt_parts.at[t, pl.ds(s * RPT, RPT)], cwork)

        def _cadd(i, _):
            sl = pl.ds(i * 16, 16)
            cvec[sl] = cvec[sl] + cwork[sl]
            return 0

        lax.fori_loop(0, RPT // 16, _cadd, 0)
    pltpu.sync_copy(cvec, cnt_hbm.at[c, s])


def _edge_aggregate(P, Q, src, dst):
    mesh = plsc.VectorSubcoreMesh(core_axis_name="c", subcore_axis_name="s",
                                  num_cores=NC, num_subcores=NS)
    f = pl.kernel(
        _edge_kernel_body,
        out_type=[
            jax.ShapeDtypeStruct((NC, NPAD, C_OUT), jnp.float32),
            jax.ShapeDtypeStruct((NC, NS, RPT), jnp.float32),
        ],
        mesh=mesh,
        compiler_params=pltpu.CompilerParams(needs_layout_passes=False),
        scratch_types=[
            pltpu.VMEM_SHARED((NPAD, C_OUT), jnp.float32),
            pltpu.VMEM_SHARED((NS, NPAD), jnp.float32),
            pltpu.VMEM((CH,), jnp.int32),
            pltpu.VMEM((CH,), jnp.int32),
            pltpu.VMEM((CH, C_OUT), jnp.float32),
            pltpu.VMEM((CH, C_OUT), jnp.float32),
            pltpu.VMEM((NPAD,), jnp.float32),
            pltpu.VMEM((RPT,), jnp.float32),
            pltpu.VMEM((RPT,), jnp.float32),
            pltpu.SemaphoreType.DMA,
        ],
    )
    return f(P, Q, src, dst)


def _combine_body(s_ref, c_ref, r_ref, w2_ref, b2_ref, g_ref, be_ref,
                  out_ref):
    S = s_ref[0][0:N, :] + s_ref[1][0:N, :]
    cnt = c_ref[0][0:N, :] + c_ref[1][0:N, :]
    mc = jnp.maximum(cnt, 1.0)
    ind = jnp.minimum(cnt, 1.0)
    agg = (jnp.dot(S / mc, w2_ref[...], preferred_element_type=jnp.float32)
           + b2_ref[...] * ind)
    o = agg + r_ref[...]
    mean = jnp.mean(o, axis=0, keepdims=True)
    var = jnp.mean((o - mean) ** 2, axis=0, keepdims=True)
    o = (o - mean) * lax.rsqrt(var + 1e-5) * g_ref[...] + be_ref[...]
    out_ref[...] = jnp.maximum(o, 0.0)


def _combine(S2, CNT2, R, W2, b2, gamma, beta):
    return pl.pallas_call(
        _combine_body,
        out_shape=jax.ShapeDtypeStruct((N, C_OUT), jnp.float32),
    )(S2, CNT2, R, W2, b2.reshape(1, C_OUT), gamma.reshape(1, C_OUT),
      beta.reshape(1, C_OUT))


def kernel(x, edge_index, edge_attr, batch, W1, b1, W2, b2, W_root,
           gamma, beta):
    src = edge_index[0]
    dst = edge_index[1]
    P, Q, R = _precompute(x, W1, b1, W_root)
    S2, CNTRAW = _edge_aggregate(P, Q, src, dst)
    CNT2 = CNTRAW.reshape(NC, NPAD, 1)
    out = _combine(S2, CNT2, R, W2, b2, gamma, beta)
    return (out, edge_index, edge_attr, batch)


# speedup vs baseline: 8.9891x; 1.5190x over previous
import functools

import jax
import jax.numpy as jnp
from jax import lax
from jax.experimental import pallas as pl
from jax.experimental.pallas import tpu as pltpu
from jax.experimental.pallas import tpu_sc as plsc

N = 10000
E = 320000
C_IN = 128
C_OUT = 128

NC = 2
NS = 16
NW = NC * NS
EPW = E // NW
CH = 80
NCHUNK = EPW // CH
NPAD = 10240
RPT = NPAD // NS


def _precompute_body(x_ref, w1_ref, b1_ref, wr_ref, p_ref, q_ref, r_ref):
    xb = x_ref[...]
    p_ref[...] = (jnp.dot(xb, w1_ref[0:C_IN, :],
                          preferred_element_type=jnp.float32) + b1_ref[...])
    q_ref[...] = jnp.dot(xb, w1_ref[C_IN:2 * C_IN, :],
                         preferred_element_type=jnp.float32)
    r_ref[...] = jnp.dot(xb, wr_ref[...], preferred_element_type=jnp.float32)


def _precompute(x, W1, b1, W_root):
    blk = 2000
    grid = N // blk
    out = jax.ShapeDtypeStruct((N, C_OUT), jnp.float32)
    return pl.pallas_call(
        _precompute_body,
        grid=(grid,),
        in_specs=[
            pl.BlockSpec((blk, C_IN), lambda i: (i, 0)),
            pl.BlockSpec((2 * C_IN, C_OUT), lambda i: (0, 0)),
            pl.BlockSpec((1, C_OUT), lambda i: (0, 0)),
            pl.BlockSpec((C_IN, C_OUT), lambda i: (0, 0)),
        ],
        out_specs=[
            pl.BlockSpec((blk, C_OUT), lambda i: (i, 0)),
            pl.BlockSpec((blk, C_OUT), lambda i: (i, 0)),
            pl.BlockSpec((blk, C_OUT), lambda i: (i, 0)),
        ],
        out_shape=[out, out, out],
    )(x, W1, b1.reshape(1, C_OUT), W_root)


NPAIR = (NCHUNK - 1) // 2


def _edge_body(p_hbm, q_hbm, src_hbm, dst_hbm, s_out, cnt_parts,
               didxA, sidxA, pbufA, didxB, sidxB, pbufB, clocal,
               gsemA, gsemB, ssemA, ssemB):
    c = lax.axis_index("c")
    s = lax.axis_index("s")
    w = c * NS + s
    ebase = w * EPW

    zvec = jnp.zeros((16,), jnp.float32)

    def _zero_row(i, _):
        for j in range(C_OUT // 16):
            pbufA[i, pl.ds(j * 16, 16)] = zvec
        return 0

    lax.fori_loop(0, CH, _zero_row, 0)

    def _zfill(k, _):
        pltpu.sync_copy(pbufA, s_out.at[pl.ds(s * RPT + k * CH, CH)])
        return 0

    lax.fori_loop(0, RPT // CH, _zfill, 0)

    def _zcnt(i, _):
        clocal[pl.ds(i * 16, 16)] = zvec
        return 0

    lax.fori_loop(0, NPAD // 16, _zcnt, 0)
    plsc.subcore_barrier()

    ones16 = jnp.ones((16,), jnp.float32)

    def _relu_rows(pbuf):
        def _relu_row(r, _):
            for t in range(C_OUT // 16):
                sl = pl.ds(t * 16, 16)
                pbuf[r, sl] = jnp.maximum(pbuf[r, sl], 0.0)
            return 0

        lax.fori_loop(0, CH, _relu_row, 0)

    def _counts(didx, p):
        for k in range(CH // 16):
            idx16 = didx[p, pl.ds(k * 16, 16)]
            plsc.addupdate_scatter(clocal, [idx16], ones16)

    def _phase(jj, nxt, didx, sidx, pbuf, gsem, ssem,
               o_didx, o_sidx, o_pbuf, o_gsem, o_cur):
        p = jnp.bitwise_and(jj, 1)
        pltpu.make_async_copy(q_hbm.at[sidx], pbuf, gsem).wait()
        _relu_rows(pbuf)
        pltpu.make_async_copy(p_hbm.at[o_didx.at[p]], o_pbuf, o_gsem).wait()
        pltpu.async_copy(q_hbm.at[o_sidx], o_pbuf, o_gsem, add=True)
        pltpu.async_copy(pbuf, s_out.at[didx.at[p]], ssem, add=True)
        _counts(didx, p)
        nb = jnp.minimum(ebase + nxt * CH, E - CH)
        pltpu.sync_copy(dst_hbm.at[pl.ds(nb, CH)], didx.at[1 - p])
        pltpu.sync_copy(src_hbm.at[pl.ds(nb, CH)], sidx)
        pltpu.make_async_copy(pbuf, s_out.at[didx.at[p]], ssem).wait()
        pltpu.async_copy(p_hbm.at[didx.at[1 - p]], pbuf, gsem)

    pltpu.sync_copy(dst_hbm.at[pl.ds(ebase, CH)], didxA.at[0])
    pltpu.sync_copy(src_hbm.at[pl.ds(ebase, CH)], sidxA)
    pltpu.async_copy(p_hbm.at[didxA.at[0]], pbufA, gsemA).wait()
    pltpu.async_copy(q_hbm.at[sidxA], pbufA, gsemA, add=True)
    pltpu.sync_copy(dst_hbm.at[pl.ds(ebase + CH, CH)], didxB.at[0])
    pltpu.sync_copy(src_hbm.at[pl.ds(ebase + CH, CH)], sidxB)
    pltpu.async_copy(p_hbm.at[didxB.at[0]], pbufB, gsemB)

    def _pair(jj, _):
        _phase(jj, 2 * jj + 2, didxA, sidxA, pbufA, gsemA, ssemA,
               didxB, sidxB, pbufB, gsemB, 2 * jj + 1)
        _phase(jj, 2 * jj + 3, didxB, sidxB, pbufB, gsemB, ssemB,
               didxA, sidxA, pbufA, gsemA, 2 * jj + 2)
        return 0

    lax.fori_loop(0, NPAIR, _pair, 0)

    pltpu.make_async_copy(q_hbm.at[sidxA], pbufA, gsemA).wait()
    _relu_rows(pbufA)
    pltpu.async_copy(pbufA, s_out.at[didxA.at[0]], ssemA, add=True)
    _counts(didxA, 0)
    pltpu.make_async_copy(pbufA, s_out.at[didxA.at[0]], ssemA).wait()
    pltpu.make_async_copy(p_hbm.at[didxB.at[0]], pbufB, gsemB).wait()

    pltpu.sync_copy(clocal, cnt_parts.at[s])
    plsc.subcore_barrier()


def _edge_kernel_body(p_hbm, q_hbm, src_hbm, dst_hbm, s_hbm, cnt_hbm,
                      s_sh, cnt_parts, didxA, sidxA, pbufA, didxB, sidxB,
                      pbufB, clocal, cwork, cvec, gsemA, gsemB, ssemA,
                      ssemB):
    c = lax.axis_index("c")
    s = lax.axis_index("s")
    _edge_body(p_hbm, q_hbm, src_hbm, dst_hbm, s_sh, cnt_parts,
               didxA, sidxA, pbufA, didxB, sidxB, pbufB, clocal,
               gsemA, gsemB, ssemA, ssemB)

    def _publish(k, _):
        base = s * RPT + k * CH
        pltpu.sync_copy(s_sh.at[pl.ds(base, CH)], pbufA)
        pltpu.sync_copy(pbufA, s_hbm.at[c, pl.ds(base, CH)])
        return 0

    lax.fori_loop(0, RPT // CH, _publish, 0)

    def _czero(i, _):
        cvec[pl.ds(i * 16, 16)] = jnp.zeros((16,), jnp.float32)
        return 0

    lax.fori_loop(0, RPT // 16, _czero, 0)
    for t in range(NS):
        pltpu.sync_copy(cnt_parts.at[t, pl.ds(s * RPT, RPT)], cwork)

        def _cadd(i, _):
            sl = pl.ds(i * 16, 16)
            cvec[sl] = cvec[sl] + cwork[sl]
            return 0

        lax.fori_loop(0, RPT // 16, _cadd, 0)
    pltpu.sync_copy(cvec, cnt_hbm.at[c, s])


def _edge_aggregate(P, Q, src, dst):
    mesh = plsc.VectorSubcoreMesh(core_axis_name="c", subcore_axis_name="s",
                                  num_cores=NC, num_subcores=NS)
    f = pl.kernel(
        _edge_kernel_body,
        out_type=[
            jax.ShapeDtypeStruct((NC, NPAD, C_OUT), jnp.float32),
            jax.ShapeDtypeStruct((NC, NS, RPT), jnp.float32),
        ],
        mesh=mesh,
        compiler_params=pltpu.CompilerParams(needs_layout_passes=False),
        scratch_types=[
            pltpu.VMEM_SHARED((NPAD, C_OUT), jnp.float32),
            pltpu.VMEM_SHARED((NS, NPAD), jnp.float32),
            pltpu.VMEM((2, CH), jnp.int32),
            pltpu.VMEM((CH,), jnp.int32),
            pltpu.VMEM((CH, C_OUT), jnp.float32),
            pltpu.VMEM((2, CH), jnp.int32),
            pltpu.VMEM((CH,), jnp.int32),
            pltpu.VMEM((CH, C_OUT), jnp.float32),
            pltpu.VMEM((NPAD,), jnp.float32),
            pltpu.VMEM((RPT,), jnp.float32),
            pltpu.VMEM((RPT,), jnp.float32),
            pltpu.SemaphoreType.DMA,
            pltpu.SemaphoreType.DMA,
            pltpu.SemaphoreType.DMA,
            pltpu.SemaphoreType.DMA,
        ],
    )
    return f(P, Q, src, dst)


def _combine_body(s_ref, c_ref, r_ref, w2_ref, b2_ref, g_ref, be_ref,
                  out_ref):
    S = s_ref[0][0:N, :] + s_ref[1][0:N, :]
    cnt = c_ref[0][0:N, :] + c_ref[1][0:N, :]
    mc = jnp.maximum(cnt, 1.0)
    ind = jnp.minimum(cnt, 1.0)
    agg = (jnp.dot(S / mc, w2_ref[...], preferred_element_type=jnp.float32)
           + b2_ref[...] * ind)
    o = agg + r_ref[...]
    mean = jnp.mean(o, axis=0, keepdims=True)
    var = jnp.mean((o - mean) ** 2, axis=0, keepdims=True)
    o = (o - mean) * lax.rsqrt(var + 1e-5) * g_ref[...] + be_ref[...]
    out_ref[...] = jnp.maximum(o, 0.0)


def _combine(S2, CNT2, R, W2, b2, gamma, beta):
    return pl.pallas_call(
        _combine_body,
        out_shape=jax.ShapeDtypeStruct((N, C_OUT), jnp.float32),
    )(S2, CNT2, R, W2, b2.reshape(1, C_OUT), gamma.reshape(1, C_OUT),
      beta.reshape(1, C_OUT))


def kernel(x, edge_index, edge_attr, batch, W1, b1, W2, b2, W_root,
           gamma, beta):
    src = edge_index[0]
    dst = edge_index[1]
    P, Q, R = _precompute(x, W1, b1, W_root)
    S2, CNTRAW = _edge_aggregate(P, Q, src, dst)
    CNT2 = CNTRAW.reshape(NC, NPAD, 1)
    out = _combine(S2, CNT2, R, W2, b2, gamma, beta)
    return (out, edge_index, edge_attr, batch)


# speedup vs baseline: 8.9961x; 1.0008x over previous
import functools

import jax
import jax.numpy as jnp
from jax import lax
from jax.experimental import pallas as pl
from jax.experimental.pallas import tpu as pltpu
from jax.experimental.pallas import tpu_sc as plsc

N = 10000
E = 320000
C_IN = 128
C_OUT = 128

NC = 2
NS = 16
NW = NC * NS
EPW = E // NW
CH = 80
NCHUNK = EPW // CH
NPAD = 10240
RPT = NPAD // NS


def _precompute_body(x_ref, w1_ref, b1_ref, wr_ref, p_ref, q_ref, r_ref):
    xb = x_ref[...]
    p_ref[...] = (jnp.dot(xb, w1_ref[0:C_IN, :],
                          preferred_element_type=jnp.float32) + b1_ref[...])
    q_ref[...] = jnp.dot(xb, w1_ref[C_IN:2 * C_IN, :],
                         preferred_element_type=jnp.float32)
    r_ref[...] = jnp.dot(xb, wr_ref[...], preferred_element_type=jnp.float32)


def _precompute(x, W1, b1, W_root):
    blk = 2000
    grid = N // blk
    out = jax.ShapeDtypeStruct((N, C_OUT), jnp.float32)
    return pl.pallas_call(
        _precompute_body,
        grid=(grid,),
        in_specs=[
            pl.BlockSpec((blk, C_IN), lambda i: (i, 0)),
            pl.BlockSpec((2 * C_IN, C_OUT), lambda i: (0, 0)),
            pl.BlockSpec((1, C_OUT), lambda i: (0, 0)),
            pl.BlockSpec((C_IN, C_OUT), lambda i: (0, 0)),
        ],
        out_specs=[
            pl.BlockSpec((blk, C_OUT), lambda i: (i, 0)),
            pl.BlockSpec((blk, C_OUT), lambda i: (i, 0)),
            pl.BlockSpec((blk, C_OUT), lambda i: (i, 0)),
        ],
        out_shape=[out, out, out],
    )(x, W1, b1.reshape(1, C_OUT), W_root)


NPAIR = (NCHUNK - 1) // 2


def _edge_body(p_hbm, q_hbm, src_hbm, dst_hbm, s_out, cnt_parts,
               didxA, sidxA, pbufA, didxB, sidxB, pbufB, clocal,
               gsemA, gsemB, ssemA, ssemB):
    c = lax.axis_index("c")
    s = lax.axis_index("s")
    w = c * NS + s
    ebase = w * EPW

    zvec = jnp.zeros((16,), jnp.float32)

    def _zero_row(i, _):
        for j in range(C_OUT // 16):
            pbufA[i, pl.ds(j * 16, 16)] = zvec
        return 0

    lax.fori_loop(0, CH, _zero_row, 0)

    def _zfill(k, _):
        pltpu.sync_copy(pbufA, s_out.at[pl.ds(s * RPT + k * CH, CH)])
        return 0

    lax.fori_loop(0, RPT // CH, _zfill, 0)

    def _zcnt(i, _):
        clocal[pl.ds(i * 16, 16)] = zvec
        return 0

    lax.fori_loop(0, NPAD // 16, _zcnt, 0)
    plsc.subcore_barrier()

    ones16 = jnp.ones((16,), jnp.float32)

    def _relu_rows(pbuf):
        @functools.partial(plsc.parallel_loop, 0, CH, unroll=2)
        def _relu_row(r):
            for t in range(C_OUT // 16):
                sl = pl.ds(t * 16, 16)
                pbuf[r, sl] = jnp.maximum(pbuf[r, sl], 0.0)

    def _counts(didx, p):
        for k in range(CH // 16):
            idx16 = didx[p, pl.ds(k * 16, 16)]
            plsc.addupdate_scatter(clocal, [idx16], ones16)

    def _phase(jj, nxt, didx, sidx, pbuf, gsem, ssem,
               o_didx, o_sidx, o_pbuf, o_gsem, o_cur):
        p = jnp.bitwise_and(jj, 1)
        pltpu.make_async_copy(q_hbm.at[sidx], pbuf, gsem).wait()
        _relu_rows(pbuf)
        pltpu.make_async_copy(p_hbm.at[o_didx.at[p]], o_pbuf, o_gsem).wait()
        pltpu.async_copy(q_hbm.at[o_sidx], o_pbuf, o_gsem, add=True)
        pltpu.async_copy(pbuf, s_out.at[didx.at[p]], ssem, add=True)
        _counts(didx, p)
        nb = jnp.minimum(ebase + nxt * CH, E - CH)
        pltpu.sync_copy(dst_hbm.at[pl.ds(nb, CH)], didx.at[1 - p])
        pltpu.sync_copy(src_hbm.at[pl.ds(nb, CH)], sidx)
        pltpu.make_async_copy(pbuf, s_out.at[didx.at[p]], ssem).wait()
        pltpu.async_copy(p_hbm.at[didx.at[1 - p]], pbuf, gsem)

    pltpu.sync_copy(dst_hbm.at[pl.ds(ebase, CH)], didxA.at[0])
    pltpu.sync_copy(src_hbm.at[pl.ds(ebase, CH)], sidxA)
    pltpu.async_copy(p_hbm.at[didxA.at[0]], pbufA, gsemA).wait()
    pltpu.async_copy(q_hbm.at[sidxA], pbufA, gsemA, add=True)
    pltpu.sync_copy(dst_hbm.at[pl.ds(ebase + CH, CH)], didxB.at[0])
    pltpu.sync_copy(src_hbm.at[pl.ds(ebase + CH, CH)], sidxB)
    pltpu.async_copy(p_hbm.at[didxB.at[0]], pbufB, gsemB)

    def _pair(jj, _):
        _phase(jj, 2 * jj + 2, didxA, sidxA, pbufA, gsemA, ssemA,
               didxB, sidxB, pbufB, gsemB, 2 * jj + 1)
        _phase(jj, 2 * jj + 3, didxB, sidxB, pbufB, gsemB, ssemB,
               didxA, sidxA, pbufA, gsemA, 2 * jj + 2)
        return 0

    lax.fori_loop(0, NPAIR, _pair, 0)

    pltpu.make_async_copy(q_hbm.at[sidxA], pbufA, gsemA).wait()
    _relu_rows(pbufA)
    pltpu.async_copy(pbufA, s_out.at[didxA.at[0]], ssemA, add=True)
    _counts(didxA, 0)
    pltpu.make_async_copy(pbufA, s_out.at[didxA.at[0]], ssemA).wait()
    pltpu.make_async_copy(p_hbm.at[didxB.at[0]], pbufB, gsemB).wait()

    pltpu.sync_copy(clocal, cnt_parts.at[s])
    plsc.subcore_barrier()


def _edge_kernel_body(p_hbm, q_hbm, src_hbm, dst_hbm, s_hbm, cnt_hbm,
                      s_sh, cnt_parts, didxA, sidxA, pbufA, didxB, sidxB,
                      pbufB, clocal, cwork, cvec, gsemA, gsemB, ssemA,
                      ssemB):
    c = lax.axis_index("c")
    s = lax.axis_index("s")
    _edge_body(p_hbm, q_hbm, src_hbm, dst_hbm, s_sh, cnt_parts,
               didxA, sidxA, pbufA, didxB, sidxB, pbufB, clocal,
               gsemA, gsemB, ssemA, ssemB)

    def _publish(k, _):
        base = s * RPT + k * CH
        pltpu.sync_copy(s_sh.at[pl.ds(base, CH)], pbufA)
        pltpu.sync_copy(pbufA, s_hbm.at[c, pl.ds(base, CH)])
        return 0

    lax.fori_loop(0, RPT // CH, _publish, 0)

    def _czero(i, _):
        cvec[pl.ds(i * 16, 16)] = jnp.zeros((16,), jnp.float32)
        return 0

    lax.fori_loop(0, RPT // 16, _czero, 0)
    for t in range(NS):
        pltpu.sync_copy(cnt_parts.at[t, pl.ds(s * RPT, RPT)], cwork)

        def _cadd(i, _):
            sl = pl.ds(i * 16, 16)
            cvec[sl] = cvec[sl] + cwork[sl]
            return 0

        lax.fori_loop(0, RPT // 16, _cadd, 0)
    pltpu.sync_copy(cvec, cnt_hbm.at[c, s])


def _edge_aggregate(P, Q, src, dst):
    mesh = plsc.VectorSubcoreMesh(core_axis_name="c", subcore_axis_name="s",
                                  num_cores=NC, num_subcores=NS)
    f = pl.kernel(
        _edge_kernel_body,
        out_type=[
            jax.ShapeDtypeStruct((NC, NPAD, C_OUT), jnp.float32),
            jax.ShapeDtypeStruct((NC, NS, RPT), jnp.float32),
        ],
        mesh=mesh,
        compiler_params=pltpu.CompilerParams(needs_layout_passes=False),
        scratch_types=[
            pltpu.VMEM_SHARED((NPAD, C_OUT), jnp.float32),
            pltpu.VMEM_SHARED((NS, NPAD), jnp.float32),
            pltpu.VMEM((2, CH), jnp.int32),
            pltpu.VMEM((CH,), jnp.int32),
            pltpu.VMEM((CH, C_OUT), jnp.float32),
            pltpu.VMEM((2, CH), jnp.int32),
            pltpu.VMEM((CH,), jnp.int32),
            pltpu.VMEM((CH, C_OUT), jnp.float32),
            pltpu.VMEM((NPAD,), jnp.float32),
            pltpu.VMEM((RPT,), jnp.float32),
            pltpu.VMEM((RPT,), jnp.float32),
            pltpu.SemaphoreType.DMA,
            pltpu.SemaphoreType.DMA,
            pltpu.SemaphoreType.DMA,
            pltpu.SemaphoreType.DMA,
        ],
    )
    return f(P, Q, src, dst)


def _combine_body(s_ref, c_ref, r_ref, w2_ref, b2_ref, g_ref, be_ref,
                  out_ref):
    S = s_ref[0][0:N, :] + s_ref[1][0:N, :]
    cnt = c_ref[0][0:N, :] + c_ref[1][0:N, :]
    mc = jnp.maximum(cnt, 1.0)
    ind = jnp.minimum(cnt, 1.0)
    agg = (jnp.dot(S / mc, w2_ref[...], preferred_element_type=jnp.float32)
           + b2_ref[...] * ind)
    o = agg + r_ref[...]
    mean = jnp.mean(o, axis=0, keepdims=True)
    var = jnp.mean((o - mean) ** 2, axis=0, keepdims=True)
    o = (o - mean) * lax.rsqrt(var + 1e-5) * g_ref[...] + be_ref[...]
    out_ref[...] = jnp.maximum(o, 0.0)


def _combine(S2, CNT2, R, W2, b2, gamma, beta):
    return pl.pallas_call(
        _combine_body,
        out_shape=jax.ShapeDtypeStruct((N, C_OUT), jnp.float32),
    )(S2, CNT2, R, W2, b2.reshape(1, C_OUT), gamma.reshape(1, C_OUT),
      beta.reshape(1, C_OUT))


def kernel(x, edge_index, edge_attr, batch, W1, b1, W2, b2, W_root,
           gamma, beta):
    src = edge_index[0]
    dst = edge_index[1]
    P, Q, R = _precompute(x, W1, b1, W_root)
    S2, CNTRAW = _edge_aggregate(P, Q, src, dst)
    CNT2 = CNTRAW.reshape(NC, NPAD, 1)
    out = _combine(S2, CNT2, R, W2, b2, gamma, beta)
    return (out, edge_index, edge_attr, batch)


# speedup vs baseline: 9.9212x; 1.1028x over previous
import functools

import jax
import jax.numpy as jnp
from jax import lax
from jax.experimental import pallas as pl
from jax.experimental.pallas import tpu as pltpu
from jax.experimental.pallas import tpu_sc as plsc

N = 10000
E = 320000
C_IN = 128
C_OUT = 128

NC = 2
NS = 16
NW = NC * NS
EPW = E // NW
CH = 80
NCHUNK = EPW // CH
NPAD = 10240
RPT = NPAD // NS


def _precompute_body(x_ref, w1_ref, b1_ref, wr_ref, p_ref, q_ref, r_ref):
    xb = x_ref[...]
    p_ref[...] = (jnp.dot(xb, w1_ref[0:C_IN, :],
                          preferred_element_type=jnp.float32) + b1_ref[...])
    q_ref[...] = jnp.dot(xb, w1_ref[C_IN:2 * C_IN, :],
                         preferred_element_type=jnp.float32)
    r_ref[...] = jnp.dot(xb, wr_ref[...], preferred_element_type=jnp.float32)


def _precompute(x, W1, b1, W_root):
    blk = 2000
    grid = N // blk
    out = jax.ShapeDtypeStruct((N, C_OUT), jnp.float32)
    return pl.pallas_call(
        _precompute_body,
        grid=(grid,),
        in_specs=[
            pl.BlockSpec((blk, C_IN), lambda i: (i, 0)),
            pl.BlockSpec((2 * C_IN, C_OUT), lambda i: (0, 0)),
            pl.BlockSpec((1, C_OUT), lambda i: (0, 0)),
            pl.BlockSpec((C_IN, C_OUT), lambda i: (0, 0)),
        ],
        out_specs=[
            pl.BlockSpec((blk, C_OUT), lambda i: (i, 0)),
            pl.BlockSpec((blk, C_OUT), lambda i: (i, 0)),
            pl.BlockSpec((blk, C_OUT), lambda i: (i, 0)),
        ],
        out_shape=[out, out, out],
    )(x, W1, b1.reshape(1, C_OUT), W_root)


NPAIR = (NCHUNK - 1) // 2
BAT = 5
NBAT = NCHUNK // BAT
NBAT_TOT = NW * NBAT


def _edge_body(p_hbm, q_hbm, idx5_hbm, s_out, cnt_parts,
               idx5, pbufA, pbufB, clocal,
               gsemA, gsemB, ssemA, ssemB):
    c = lax.axis_index("c")
    s = lax.axis_index("s")
    w = c * NS + s

    zvec = jnp.zeros((16,), jnp.float32)

    def _zero_row(i, _):
        for j in range(C_OUT // 16):
            pbufA[i, pl.ds(j * 16, 16)] = zvec
        return 0

    lax.fori_loop(0, CH, _zero_row, 0)

    def _zfill(k, _):
        pltpu.sync_copy(pbufA, s_out.at[pl.ds(s * RPT + k * CH, CH)])
        return 0

    lax.fori_loop(0, RPT // CH, _zfill, 0)

    def _zcnt(i, _):
        clocal[pl.ds(i * 16, 16)] = zvec
        return 0

    lax.fori_loop(0, NPAD // 16, _zcnt, 0)
    plsc.subcore_barrier()

    ones16 = jnp.ones((16,), jnp.float32)

    def _relu_rows(pbuf):
        def _relu_row(r, _):
            for t in range(C_OUT // 16):
                sl = pl.ds(t * 16, 16)
                pbuf[r, sl] = jnp.maximum(pbuf[r, sl], 0.0)
            return 0

        lax.fori_loop(0, CH, _relu_row, 0)

    def _counts(hf, sl):
        for k in range(CH // 16):
            idx16 = idx5[hf, sl, 0, pl.ds(k * 16, 16)]
            plsc.addupdate_scatter(clocal, [idx16], ones16)

    bbase = w * NBAT

    def _didx(cl):
        b = cl // BAT
        return idx5.at[jnp.bitwise_and(b, 1), cl - b * BAT, 0]

    def _sidx(cl):
        b = cl // BAT
        return idx5.at[jnp.bitwise_and(b, 1), cl - b * BAT, 1]

    def _refill(cl):
        b = cl // BAT

        @pl.when(cl == b * BAT)
        def _():
            nb = jnp.minimum(bbase + b + 1, NBAT_TOT - 1)
            pltpu.sync_copy(idx5_hbm.at[nb],
                            idx5.at[jnp.bitwise_and(b + 1, 1)])

    def _phase(cl, pbuf, gsem, ssem, o_pbuf, o_gsem):
        b = cl // BAT
        hf = jnp.bitwise_and(b, 1)
        sl = cl - b * BAT
        pltpu.make_async_copy(q_hbm.at[_sidx(cl)], pbuf, gsem).wait()
        _relu_rows(pbuf)
        pltpu.make_async_copy(p_hbm.at[_didx(cl + 1)], o_pbuf,
                              o_gsem).wait()
        pltpu.async_copy(q_hbm.at[_sidx(cl + 1)], o_pbuf, o_gsem, add=True)
        pltpu.async_copy(pbuf, s_out.at[_didx(cl)], ssem, add=True)
        _counts(hf, sl)
        _refill(cl)
        pltpu.make_async_copy(pbuf, s_out.at[_didx(cl)], ssem).wait()
        pltpu.async_copy(p_hbm.at[_didx(cl + 2)], pbuf, gsem)

    pltpu.sync_copy(idx5_hbm.at[bbase], idx5.at[0])
    pltpu.async_copy(p_hbm.at[_didx(0)], pbufA, gsemA).wait()
    pltpu.async_copy(q_hbm.at[_sidx(0)], pbufA, gsemA, add=True)
    pltpu.async_copy(p_hbm.at[_didx(1)], pbufB, gsemB)

    def _pair(jj, _):
        _phase(2 * jj, pbufA, gsemA, ssemA, pbufB, gsemB)
        _phase(2 * jj + 1, pbufB, gsemB, ssemB, pbufA, gsemA)
        return 0

    lax.fori_loop(0, NPAIR, _pair, 0)

    last = NCHUNK - 1
    pltpu.make_async_copy(q_hbm.at[_sidx(last)], pbufA, gsemA).wait()
    _relu_rows(pbufA)
    pltpu.async_copy(pbufA, s_out.at[_didx(last)], ssemA, add=True)
    _counts(jnp.bitwise_and(last // BAT, 1), last % BAT)
    pltpu.make_async_copy(pbufA, s_out.at[_didx(last)], ssemA).wait()
    pltpu.make_async_copy(p_hbm.at[_didx(last)], pbufB, gsemB).wait()

    pltpu.sync_copy(clocal, cnt_parts.at[s])
    plsc.subcore_barrier()


def _edge_kernel_body(p_hbm, q_hbm, idx5_hbm, s_hbm, cnt_hbm,
                      s_sh, cnt_parts, idx5, pbufA, pbufB,
                      clocal, cwork, cvec, gsemA, gsemB, ssemA, ssemB):
    c = lax.axis_index("c")
    s = lax.axis_index("s")
    _edge_body(p_hbm, q_hbm, idx5_hbm, s_sh, cnt_parts,
               idx5, pbufA, pbufB, clocal,
               gsemA, gsemB, ssemA, ssemB)

    def _publish(k, _):
        base = s * RPT + k * CH
        pltpu.sync_copy(s_sh.at[pl.ds(base, CH)], pbufA)
        pltpu.sync_copy(pbufA, s_hbm.at[c, pl.ds(base, CH)])
        return 0

    lax.fori_loop(0, RPT // CH, _publish, 0)

    def _czero(i, _):
        cvec[pl.ds(i * 16, 16)] = jnp.zeros((16,), jnp.float32)
        return 0

    lax.fori_loop(0, RPT // 16, _czero, 0)
    for t in range(NS):
        pltpu.sync_copy(cnt_parts.at[t, pl.ds(s * RPT, RPT)], cwork)

        def _cadd(i, _):
            sl = pl.ds(i * 16, 16)
            cvec[sl] = cvec[sl] + cwork[sl]
            return 0

        lax.fori_loop(0, RPT // 16, _cadd, 0)
    pltpu.sync_copy(cvec, cnt_hbm.at[c, s])


def _edge_aggregate(P, Q, idx5_in):
    mesh = plsc.VectorSubcoreMesh(core_axis_name="c", subcore_axis_name="s",
                                  num_cores=NC, num_subcores=NS)
    f = pl.kernel(
        _edge_kernel_body,
        out_type=[
            jax.ShapeDtypeStruct((NC, NPAD, C_OUT), jnp.float32),
            jax.ShapeDtypeStruct((NC, NS, RPT), jnp.float32),
        ],
        mesh=mesh,
        compiler_params=pltpu.CompilerParams(needs_layout_passes=False),
        scratch_types=[
            pltpu.VMEM_SHARED((NPAD, C_OUT), jnp.float32),
            pltpu.VMEM_SHARED((NS, NPAD), jnp.float32),
            pltpu.VMEM((2, BAT, 2, CH), jnp.int32),
            pltpu.VMEM((CH, C_OUT), jnp.float32),
            pltpu.VMEM((CH, C_OUT), jnp.float32),
            pltpu.VMEM((NPAD,), jnp.float32),
            pltpu.VMEM((RPT,), jnp.float32),
            pltpu.VMEM((RPT,), jnp.float32),
            pltpu.SemaphoreType.DMA,
            pltpu.SemaphoreType.DMA,
            pltpu.SemaphoreType.DMA,
            pltpu.SemaphoreType.DMA,
        ],
    )
    return f(P, Q, idx5_in)


def _combine_body(s_ref, c_ref, r_ref, w2_ref, b2_ref, g_ref, be_ref,
                  out_ref):
    S = s_ref[0][0:N, :] + s_ref[1][0:N, :]
    cnt = c_ref[0][0:N, :] + c_ref[1][0:N, :]
    mc = jnp.maximum(cnt, 1.0)
    ind = jnp.minimum(cnt, 1.0)
    agg = (jnp.dot(S / mc, w2_ref[...], preferred_element_type=jnp.float32)
           + b2_ref[...] * ind)
    o = agg + r_ref[...]
    mean = jnp.mean(o, axis=0, keepdims=True)
    var = jnp.mean((o - mean) ** 2, axis=0, keepdims=True)
    o = (o - mean) * lax.rsqrt(var + 1e-5) * g_ref[...] + be_ref[...]
    out_ref[...] = jnp.maximum(o, 0.0)


def _combine(S2, CNT2, R, W2, b2, gamma, beta):
    return pl.pallas_call(
        _combine_body,
        out_shape=jax.ShapeDtypeStruct((N, C_OUT), jnp.float32),
    )(S2, CNT2, R, W2, b2.reshape(1, C_OUT), gamma.reshape(1, C_OUT),
      beta.reshape(1, C_OUT))


def kernel(x, edge_index, edge_attr, batch, W1, b1, W2, b2, W_root,
           gamma, beta):
    src = edge_index[0]
    dst = edge_index[1]
    idx5_in = jnp.concatenate(
        [dst.reshape(NBAT_TOT, BAT, 1, CH), src.reshape(NBAT_TOT, BAT, 1, CH)],
        axis=2)
    P, Q, R = _precompute(x, W1, b1, W_root)
    S2, CNTRAW = _edge_aggregate(P, Q, idx5_in)
    CNT2 = CNTRAW.reshape(NC, NPAD, 1)
    out = _combine(S2, CNT2, R, W2, b2, gamma, beta)
    return (out, edge_index, edge_attr, batch)
